# baseline (speedup 1.0000x reference)
import functools

import jax
import jax.numpy as jnp
import numpy as np
from jax import lax
from jax.experimental import pallas as pl
from jax.experimental.pallas import tpu as pltpu
from jax.experimental.pallas import tpu_sc as plsc

N = 10000
E = 320000
D_IN = 128
D = 256
H = 8
DH = 32
L = 4
DFF = 1024
DE = 16
NG = 64

NC = 2
NS = 16
NP = 2
HP = H // (NC * NP)
DP = D // (NC * NP)
EPT = E // NS
CB = 80
NCHUNK = EPT // CB
ACC_W = DP + 16

BR = 2000
NB = N // BR
INV_SQRT_DH = 1.0 / np.sqrt(DH)


def _ln(h, g, b):
    m = h.mean(-1, keepdims=True)
    v = ((h - m) ** 2).mean(-1, keepdims=True)
    return (h - m) / jnp.sqrt(v + 1e-5) * g + b


def _dot(a, b):
    return jnp.dot(a, b, preferred_element_type=jnp.float32)


def _inproj_body(x_ref, w_ref, b_ref, g_ref, bb_ref, o_ref):
    h = _dot(x_ref[...], w_ref[...]) + b_ref[...]
    o_ref[...] = jax.nn.gelu(_ln(h, g_ref[...], bb_ref[...]))


def _inproj(x, Win, b_in, g, b):
    return pl.pallas_call(
        _inproj_body,
        grid=(NB,),
        in_specs=[
            pl.BlockSpec((BR, D_IN), lambda i: (i, 0)),
            pl.BlockSpec((D_IN, D), lambda i: (0, 0)),
            pl.BlockSpec((1, D), lambda i: (0, 0)),
            pl.BlockSpec((1, D), lambda i: (0, 0)),
            pl.BlockSpec((1, D), lambda i: (0, 0)),
        ],
        out_specs=pl.BlockSpec((BR, D), lambda i: (i, 0)),
        out_shape=jax.ShapeDtypeStruct((N, D), jnp.float32),
    )(x, Win, b_in.reshape(1, D), g.reshape(1, D), b.reshape(1, D))


def _qkv_body(h_ref, wq_ref, wk_ref, wv_ref, q_ref, kv_ref):
    h = h_ref[...]
    q_ref[0] = _dot(h, wq_ref[0])
    kb = lax.bitcast_convert_type(
        _dot(h, wk_ref[0]).astype(jnp.bfloat16), jnp.uint16).astype(jnp.int32)
    vb = lax.bitcast_convert_type(
        _dot(h, wv_ref[0]).astype(jnp.bfloat16), jnp.uint16).astype(jnp.int32)
    kv_ref[0] = kb | (vb << 16)


def _qkv(h, Wq, Wk, Wv):
    q4, kv4 = pl.pallas_call(
        _qkv_body,
        grid=(NC * NP, NB),
        in_specs=[
            pl.BlockSpec((BR, D), lambda q, i: (i, 0)),
            pl.BlockSpec((1, D, DP), lambda q, i: (q, 0, 0)),
            pl.BlockSpec((1, D, DP), lambda q, i: (q, 0, 0)),
            pl.BlockSpec((1, D, DP), lambda q, i: (q, 0, 0)),
        ],
        out_specs=[
            pl.BlockSpec((1, BR, DP), lambda q, i: (q, i, 0)),
            pl.BlockSpec((1, BR, DP), lambda q, i: (q, i, 0)),
        ],
        out_shape=[
            jax.ShapeDtypeStruct((NC * NP, N, DP), jnp.float32),
            jax.ShapeDtypeStruct((NC * NP, N, DP), jnp.int32),
        ],
    )(h, Wq, Wk, Wv)
    return q4.reshape(NC * NP * N, DP), kv4.reshape(NC * NP * N, DP)


_EB_BLK = 20000


def _eb_body(ea_ref, we_ref, be_ref, o_ref):
    o_ref[0] = _dot(ea_ref[...], we_ref[0]) + be_ref[0]


def _edge_bias(edge_attr, We, be):
    return pl.pallas_call(
        _eb_body,
        grid=(L, E // _EB_BLK),
        in_specs=[
            pl.BlockSpec((_EB_BLK, DE), lambda l, i: (i, 0)),
            pl.BlockSpec((1, DE, H), lambda l, i: (l, 0, 0)),
            pl.BlockSpec((1, 1, H), lambda l, i: (l, 0, 0)),
        ],
        out_specs=pl.BlockSpec((1, _EB_BLK, H), lambda l, i: (l, i, 0)),
        out_shape=jax.ShapeDtypeStruct((L, E, H), jnp.float32),
    )(edge_attr, We, be.reshape(L, 1, H))


def _outffn_body(h_ref, m0_ref, m1_ref, wo_ref, bo_ref, g1_ref, b1_ref,
                 w1_ref, bf1_ref, w2_ref, bf2_ref, g2_ref, b2_ref, o_ref):
    m0 = m0_ref[...]
    m1 = m1_ref[...]
    msg = jnp.concatenate([m0[0], m1[0], m0[1], m1[1]], axis=-1)
    h1 = _ln(h_ref[...] + _dot(msg, wo_ref[...]) + bo_ref[...],
             g1_ref[...], b1_ref[...])
    ffn = _dot(jax.nn.gelu(_dot(h1, w1_ref[...]) + bf1_ref[...]),
               w2_ref[...]) + bf2_ref[...]
    o_ref[...] = _ln(h1 + ffn, g2_ref[...], b2_ref[...])


def _outffn(h, msg0, msg1, Wo, bo, g1, b1, W1, bf1, W2, bf2, g2, b2):
    r1 = lambda a, n: a.reshape(1, n)
    return pl.pallas_call(
        _outffn_body,
        grid=(NB,),
        in_specs=[
            pl.BlockSpec((BR, D), lambda i: (i, 0)),
            pl.BlockSpec((NC, BR, DP), lambda i: (0, i, 0)),
            pl.BlockSpec((NC, BR, DP), lambda i: (0, i, 0)),
            pl.BlockSpec((D, D), lambda i: (0, 0)),
            pl.BlockSpec((1, D), lambda i: (0, 0)),
            pl.BlockSpec((1, D), lambda i: (0, 0)),
            pl.BlockSpec((1, D), lambda i: (0, 0)),
            pl.BlockSpec((D, DFF), lambda i: (0, 0)),
            pl.BlockSpec((1, DFF), lambda i: (0, 0)),
            pl.BlockSpec((DFF, D), lambda i: (0, 0)),
            pl.BlockSpec((1, D), lambda i: (0, 0)),
            pl.BlockSpec((1, D), lambda i: (0, 0)),
            pl.BlockSpec((1, D), lambda i: (0, 0)),
        ],
        out_specs=pl.BlockSpec((BR, D), lambda i: (i, 0)),
        out_shape=jax.ShapeDtypeStruct((N, D), jnp.float32),
    )(h, msg0.reshape(NC, N, DP), msg1.reshape(NC, N, DP), Wo, r1(bo, D),
      r1(g1, D), r1(b1, D), W1, r1(bf1, DFF), W2, r1(bf2, D), r1(g2, D),
      r1(b2, D))


def _pool_body(h_ref, batch_ref, wh_ref, bh_ref, wm_ref, bm_ref,
               pred_ref, mean_ref, sums, cnts):
    i = pl.program_id(0)

    @pl.when(i == 0)
    def _():
        sums[...] = jnp.zeros_like(sums)
        cnts[...] = jnp.zeros_like(cnts)

    b = batch_ref[0, 0]
    mask = (b[:, None] == lax.broadcasted_iota(jnp.int32, (BR, NG), 1)
            ).astype(jnp.float32)
    sums[...] += lax.dot_general(mask, h_ref[...], (((0,), (0,)), ((), ())),
                                 preferred_element_type=jnp.float32)
    cnts[...] += lax.dot_general(mask, jnp.ones((BR, D), jnp.float32),
                                 (((0,), (0,)), ((), ())),
                                 preferred_element_type=jnp.float32)

    @pl.when(i == NB - 1)
    def _():
        x_mean = sums[...] / jnp.maximum(cnts[...], 1.0)
        hh = jax.nn.gelu(_dot(x_mean, wh_ref[...]) + bh_ref[...])
        pred_ref[...] = _dot(hh, wm_ref[...]) + bm_ref[...]
        mean_ref[...] = x_mean


def _pool_head(h, batch, Wh1, bh1, Wml, bml):
    return pl.pallas_call(
        _pool_body,
        grid=(NB,),
        in_specs=[
            pl.BlockSpec((BR, D), lambda i: (i, 0)),
            pl.BlockSpec((1, 1, BR), lambda i: (i, 0, 0)),
            pl.BlockSpec((D, D // 2), lambda i: (0, 0)),
            pl.BlockSpec((1, D // 2), lambda i: (0, 0)),
            pl.BlockSpec((D // 2, 2), lambda i: (0, 0)),
            pl.BlockSpec((1, 2), lambda i: (0, 0)),
        ],
        out_specs=[
            pl.BlockSpec((NG, 2), lambda i: (0, 0)),
            pl.BlockSpec((NG, D), lambda i: (0, 0)),
        ],
        out_shape=[
            jax.ShapeDtypeStruct((NG, 2), jnp.float32),
            jax.ShapeDtypeStruct((NG, D), jnp.float32),
        ],
        scratch_shapes=[
            pltpu.VMEM((NG, D), jnp.float32),
            pltpu.VMEM((NG, D), jnp.float32),
        ],
    )(h, batch.reshape(NB, 1, BR), Wh1, bh1.reshape(1, D // 2),
      Wml, bml.reshape(1, 2))


def _sc_mesh():
    return plsc.VectorSubcoreMesh(core_axis_name="c", subcore_axis_name="s",
                                  num_cores=NC, num_subcores=NS)


def _iota16():
    return lax.iota(jnp.int32, 16)


def _full16(x):
    return jnp.zeros((16,), jnp.int32) + x


def _edge_chunk_compute(qd, kvs, ebb, exv, c, p):
    iota = _iota16()

    def _group(g, _):
        rows = g * 16 + iota

        def _kcol(cc):
            return plsc.bitcast(plsc.load_gather(kvs, [rows, cc]) << 16,
                                jnp.float32)

        def _qk(j, acc):
            a00, a01, a10, a11 = acc
            c00 = (iota + j) & (DH - 1)
            c01 = (iota + (DH // 2 + j)) & (DH - 1)
            c10 = DH + c00
            c11 = DH + c01
            a00 = a00 + plsc.load_gather(qd, [rows, c00]) * _kcol(c00)
            a01 = a01 + plsc.load_gather(qd, [rows, c01]) * _kcol(c01)
            a10 = a10 + plsc.load_gather(qd, [rows, c10]) * _kcol(c10)
            a11 = a11 + plsc.load_gather(qd, [rows, c11]) * _kcol(c11)
            return a00, a01, a10, a11

        zf = jnp.zeros((16,), jnp.float32)
        a00, a01, a10, a11 = plsc.parallel_loop(
            0, DH // 2, unroll=8, carry=(zf, zf, zf, zf))(_qk)
        ebbase = c * (NP * HP) + p * HP
        eb0 = plsc.load_gather(ebb, [rows, _full16(ebbase)])
        eb1 = plsc.load_gather(ebb, [rows, _full16(ebbase + 1)])
        ex0 = jnp.exp((a00 + a01) * INV_SQRT_DH + eb0)
        ex1 = jnp.exp((a10 + a11) * INV_SQRT_DH + eb1)
        plsc.store_scatter(exv, [rows, _full16(DP)], ex0)
        plsc.store_scatter(exv, [rows, _full16(DP + 1)], ex1)

        vmask = jnp.full((16,), -65536, jnp.int32)

        def _ev(j):
            c0 = (iota + j) & (DH - 1)
            c1 = DH + c0
            v0 = plsc.bitcast(plsc.load_gather(kvs, [rows, c0]) & vmask,
                              jnp.float32)
            v1 = plsc.bitcast(plsc.load_gather(kvs, [rows, c1]) & vmask,
                              jnp.float32)
            plsc.store_scatter(exv, [rows, c0], v0 * ex0)
            plsc.store_scatter(exv, [rows, c1], v1 * ex1)

        plsc.parallel_loop(0, DH, unroll=8)(_ev)
        return 0

    lax.fori_loop(0, CB // 16, _group, 0)


def _edge_kernel(p, q_hbm, kv_hbm, src_hbm, dst_hbm, eb_hbm, out_hbm,
                 msgacc,
                 srcA, dstrA, dstaA, qdA, kvsA, ebA, exvA,
                 srcB, dstrB, dstaB, qdB, kvsB, ebB, exvB,
                 srcC, dstrC, dstaC, qdC, kvsC, ebC, exvC,
                 zbuf, macc, obuf, macc2, obuf2,
                 gA, gB, gC, iA, iB, iC, scA, scB, scC):
    c = lax.axis_index("c")
    s = lax.axis_index("s")
    qoff = (c * NP + p) * N
    iota = _iota16()
    zf = jnp.zeros((16,), jnp.float32)

    def _zrow(i, _):
        for j in range(ACC_W // 16):
            plsc.store_scatter(zbuf, [_full16(i), _full16(j * 16) + iota], zf)
        return 0
    lax.fori_loop(0, 16, _zrow, 0)

    def _zacc(t, _):
        pltpu.async_copy(zbuf, msgacc.at[pl.ds(s * 624 + t * 16, 16)], gA)
        return 0
    lax.fori_loop(0, 39, _zacc, 0)

    @pl.when(s == NS - 1)
    def _():
        pltpu.async_copy(zbuf, msgacc.at[pl.ds(9984, 16)], gA)

    def _zacc_wait(t, _):
        pltpu.make_async_copy(zbuf, msgacc.at[pl.ds(s * 624, 16)], gA).wait()
        return 0
    lax.fori_loop(0, 39, _zacc_wait, 0)

    @pl.when(s == NS - 1)
    def _():
        pltpu.make_async_copy(zbuf, msgacc.at[pl.ds(9984, 16)], gA).wait()

    def _zpad(exv):
        def body(r, _):
            plsc.store_scatter(exv, [_full16(r), _full16(DP) + iota], zf,
                               mask=iota >= HP)
            return 0
        lax.fori_loop(0, CB, body, 0)
    _zpad(exvA)
    _zpad(exvB)
    _zpad(exvC)

    plsc.subcore_barrier()

    e0 = s * EPT
    LAST = NCHUNK - 1
    sets = (
        (srcA, dstrA, dstaA, qdA, kvsA, ebA, exvA, gA, iA, scA),
        (srcB, dstrB, dstaB, qdB, kvsB, ebB, exvB, gB, iB, scB),
        (srcC, dstrC, dstaC, qdC, kvsC, ebC, exvC, gC, iC, scC),
    )

    def _issue_idx(t, st):
        (srcv, dstr, dsta, qd, kvs, eb, exv, g, si, sc) = st
        base = e0 + t * CB
        pltpu.async_copy(src_hbm.at[pl.ds(base, CB)], srcv, si)
        pltpu.async_copy(dst_hbm.at[pl.ds(base, CB)], dstr, si)
        pltpu.async_copy(eb_hbm.at[pl.ds(base, CB)], eb, si)

    def _wait_idx(t, st):
        (srcv, dstr, dsta, qd, kvs, eb, exv, g, si, sc) = st
        base = e0 + t * CB
        pltpu.make_async_copy(src_hbm.at[pl.ds(base, CB)], srcv, si).wait()
        pltpu.make_async_copy(dst_hbm.at[pl.ds(base, CB)], dstr, si).wait()
        pltpu.make_async_copy(eb_hbm.at[pl.ds(base, CB)], eb, si).wait()

    def _issue_gathers(st):
        (srcv, dstr, dsta, qd, kvs, eb, exv, g, si, sc) = st
        for gi in range(CB // 16):
            sl = pl.ds(gi * 16, 16)
            dsta[sl] = dstr[sl] + qoff
            srcv[sl] = srcv[sl] + qoff
        pltpu.async_copy(q_hbm.at[dsta], qd, g)
        pltpu.async_copy(kv_hbm.at[srcv], kvs, g)

    def _wait_gathers(st):
        (srcv, dstr, dsta, qd, kvs, eb, exv, g, si, sc) = st
        pltpu.make_async_copy(q_hbm.at[dsta], qd, g).wait()
        pltpu.make_async_copy(kv_hbm.at[srcv], kvs, g).wait()

    def _wait_scatter(st):
        (srcv, dstr, dsta, qd, kvs, eb, exv, g, si, sc) = st
        pltpu.make_async_copy(exv, msgacc.at[dstr], sc).wait()

    def _step(t, sc_cur, sn1, sn2):
        @pl.when(t + 1 <= LAST)
        def _():
            _wait_idx(t + 1, sn1)
            _issue_gathers(sn1)

        @pl.when(jnp.logical_and(t >= 1, t + 2 <= LAST))
        def _():
            _wait_scatter(sn2)

        @pl.when(t + 2 <= LAST)
        def _():
            _issue_idx(t + 2, sn2)

        st = sc_cur
        _wait_gathers(st)
        (srcv, dstr, dsta, qd, kvs, eb, exv, g, si, sc) = st
        _edge_chunk_compute(qd, kvs, eb, exv, c, p)
        pltpu.async_copy(exv, msgacc.at[dstr], sc, add=True)

    _issue_idx(0, sets[0])
    _wait_idx(0, sets[0])
    _issue_gathers(sets[0])
    _issue_idx(1, sets[1])

    def _pipe(i, _):
        t0 = 3 * i
        _step(t0, sets[0], sets[1], sets[2])
        _step(t0 + 1, sets[1], sets[2], sets[0])
        _step(t0 + 2, sets[2], sets[0], sets[1])
        return 0

    lax.fori_loop(0, NCHUNK // 3, _pipe, 0)
    for r in range(NCHUNK - (NCHUNK // 3) * 3):
        t = (NCHUNK // 3) * 3 + r
        _step(t, sets[t % 3], sets[(t + 1) % 3], sets[(t + 2) % 3])

    for st in sets:
        _wait_scatter(st)

    plsc.subcore_barrier()

    nblk = 39 + jnp.where(s == NS - 1, 1, 0)

    def _r0(b):
        return jnp.where(b < 39, s * 624 + b * 16, 9984)

    pars = ((macc, obuf, iA, scA), (macc2, obuf2, iB, scB))

    def _nstep(b, cur, nxt):
        (macc_c, obuf_c, si_c, so_c) = cur
        (macc_n, obuf_n, si_n, so_n) = nxt

        @pl.when(b + 1 < nblk)
        def _():
            pltpu.async_copy(msgacc.at[pl.ds(_r0(b + 1), 16)], macc_n, si_n)

        pltpu.make_async_copy(msgacc.at[pl.ds(0, 16)], macc_c, si_c).wait()

        @pl.when(b >= 2)
        def _():
            pltpu.make_async_copy(obuf_c, out_hbm.at[pl.ds(0, 16)],
                                  so_c).wait()

        def _row(r, _):
            rr = _full16(r)
            for j in range(DP // 16):
                den = plsc.load_gather(macc_c, [rr, _full16(DP + j // 2)])
                val = plsc.load_gather(macc_c, [rr, _full16(j * 16) + iota])
                plsc.store_scatter(obuf_c, [rr, _full16(j * 16) + iota],
                                   val / (den + 1e-9))
            return 0
        lax.fori_loop(0, 16, _row, 0)
        pltpu.async_copy(obuf_c, out_hbm.at[pl.ds(c * N + _r0(b), 16)], so_c)

    pltpu.async_copy(msgacc.at[pl.ds(_r0(0), 16)], macc, iA)

    def _npair(k, _):
        b0 = 2 * k

        @pl.when(b0 < nblk)
        def _():
            _nstep(b0, pars[0], pars[1])

        @pl.when(b0 + 1 < nblk)
        def _():
            _nstep(b0 + 1, pars[1], pars[0])
        return 0
    lax.fori_loop(0, 20, _npair, 0)

    for (_m, obuf_c, _si, so_c) in pars:
        pltpu.make_async_copy(obuf_c, out_hbm.at[pl.ds(0, 16)], so_c).wait()


def _edge_attention(p, q4, kv4, src, dst, eb_l):
    f32 = jnp.float32
    scratch = [pltpu.VMEM_SHARED((N, ACC_W), f32)]
    for _ in range(3):
        scratch += [
            pltpu.VMEM((CB,), jnp.int32),
            pltpu.VMEM((CB,), jnp.int32),
            pltpu.VMEM((CB,), jnp.int32),
            pltpu.VMEM((CB, DP), f32),
            pltpu.VMEM((CB, DP), jnp.int32),
            pltpu.VMEM((CB, H), f32),
            pltpu.VMEM((CB, ACC_W), f32),
        ]
    scratch += [
        pltpu.VMEM((16, ACC_W), f32),
        pltpu.VMEM((16, ACC_W), f32),
        pltpu.VMEM((16, DP), f32),
        pltpu.VMEM((16, ACC_W), f32),
        pltpu.VMEM((16, DP), f32),
    ] + [pltpu.SemaphoreType.DMA] * 9

    run = pl.kernel(
        functools.partial(_edge_kernel, p),
        out_type=jax.ShapeDtypeStruct((NC * N, DP), f32),
        mesh=_sc_mesh(),
        scratch_types=scratch,
        compiler_params=pltpu.CompilerParams(use_tc_tiling_on_sc=False,
                                             needs_layout_passes=False),
    )
    return run(q4, kv4, src, dst, eb_l)


def _quarters(W):
    return W.reshape(D, NC * NP, DP).transpose(1, 0, 2)


def kernel(x, edge_index, batch, edge_attr, Win, b_in, ln0_g, ln0_b,
           Wq, Wk, Wv, Wo, bo, We, be, ln1_g, ln1_b,
           W1, b1, W2, b2, ln2_g, ln2_b,
           Wh1, bh1, Wm, bm, Wlv, blv):
    src = edge_index[0]
    dst = edge_index[1]

    h = _inproj(x, Win, b_in, ln0_g, ln0_b)
    eb_all = _edge_bias(edge_attr, We, be)

    for l in range(L):
        q4, kv4 = _qkv(h, _quarters(Wq[l]), _quarters(Wk[l]),
                       _quarters(Wv[l]))
        msg0 = _edge_attention(0, q4, kv4, src, dst, eb_all[l])
        msg1 = _edge_attention(1, q4, kv4, src, dst, eb_all[l])
        h = _outffn(h, msg0, msg1, Wo[l], bo[l],
                    ln1_g[l], ln1_b[l], W1[l], b1[l], W2[l], b2[l],
                    ln2_g[l], ln2_b[l])

    Wml = jnp.concatenate([Wm, Wlv], axis=1)
    bml = jnp.concatenate([bm, blv], axis=0)
    pred, x_mean = _pool_head(h, batch, Wh1, bh1, Wml, bml)
    return (pred, x_mean)


# speedup vs baseline: 1.0298x; 1.0298x over previous
import functools

import jax
import jax.numpy as jnp
import numpy as np
from jax import lax
from jax.experimental import pallas as pl
from jax.experimental.pallas import tpu as pltpu
from jax.experimental.pallas import tpu_sc as plsc

N = 10000
E = 320000
D_IN = 128
D = 256
H = 8
DH = 32
L = 4
DFF = 1024
DE = 16
NG = 64

NC = 2
NS = 16
NP = 2
HP = H // (NC * NP)
DP = D // (NC * NP)
EPT = E // NS
CB = 80
NCHUNK = EPT // CB
ACC_W = DP + 16

BR = 2000
NB = N // BR
INV_SQRT_DH = 1.0 / np.sqrt(DH)


def _ln(h, g, b):
    m = h.mean(-1, keepdims=True)
    v = ((h - m) ** 2).mean(-1, keepdims=True)
    return (h - m) / jnp.sqrt(v + 1e-5) * g + b


def _dot(a, b):
    return jnp.dot(a, b, preferred_element_type=jnp.float32)


def _inproj_body(x_ref, w_ref, b_ref, g_ref, bb_ref, o_ref):
    h = _dot(x_ref[...], w_ref[...]) + b_ref[...]
    o_ref[...] = jax.nn.gelu(_ln(h, g_ref[...], bb_ref[...]))


def _inproj(x, Win, b_in, g, b):
    return pl.pallas_call(
        _inproj_body,
        grid=(NB,),
        in_specs=[
            pl.BlockSpec((BR, D_IN), lambda i: (i, 0)),
            pl.BlockSpec((D_IN, D), lambda i: (0, 0)),
            pl.BlockSpec((1, D), lambda i: (0, 0)),
            pl.BlockSpec((1, D), lambda i: (0, 0)),
            pl.BlockSpec((1, D), lambda i: (0, 0)),
        ],
        out_specs=pl.BlockSpec((BR, D), lambda i: (i, 0)),
        out_shape=jax.ShapeDtypeStruct((N, D), jnp.float32),
    )(x, Win, b_in.reshape(1, D), g.reshape(1, D), b.reshape(1, D))


def _qkv_body(h_ref, wq_ref, wk_ref, wv_ref, q_ref, kv_ref):
    h = h_ref[...]
    q_ref[0] = _dot(h, wq_ref[0])
    kv_ref[0] = jnp.concatenate([_dot(h, wk_ref[0]), _dot(h, wv_ref[0])],
                                axis=-1)


def _qkv(h, Wq, Wk, Wv):
    q4, kv4 = pl.pallas_call(
        _qkv_body,
        grid=(NC * NP, NB),
        in_specs=[
            pl.BlockSpec((BR, D), lambda q, i: (i, 0)),
            pl.BlockSpec((1, D, DP), lambda q, i: (q, 0, 0)),
            pl.BlockSpec((1, D, DP), lambda q, i: (q, 0, 0)),
            pl.BlockSpec((1, D, DP), lambda q, i: (q, 0, 0)),
        ],
        out_specs=[
            pl.BlockSpec((1, BR, DP), lambda q, i: (q, i, 0)),
            pl.BlockSpec((1, BR, 2 * DP), lambda q, i: (q, i, 0)),
        ],
        out_shape=[
            jax.ShapeDtypeStruct((NC * NP, N, DP), jnp.float32),
            jax.ShapeDtypeStruct((NC * NP, N, 2 * DP), jnp.float32),
        ],
    )(h, Wq, Wk, Wv)
    return q4.reshape(NC * NP * N, DP), kv4.reshape(NC * NP * N, 2 * DP)


_EB_BLK = 20000


def _eb_body(ea_ref, we_ref, be_ref, o_ref):
    o_ref[0] = _dot(ea_ref[...], we_ref[0]) + be_ref[0]


def _edge_bias(edge_attr, We, be):
    return pl.pallas_call(
        _eb_body,
        grid=(L, E // _EB_BLK),
        in_specs=[
            pl.BlockSpec((_EB_BLK, DE), lambda l, i: (i, 0)),
            pl.BlockSpec((1, DE, H), lambda l, i: (l, 0, 0)),
            pl.BlockSpec((1, 1, H), lambda l, i: (l, 0, 0)),
        ],
        out_specs=pl.BlockSpec((1, _EB_BLK, H), lambda l, i: (l, i, 0)),
        out_shape=jax.ShapeDtypeStruct((L, E, H), jnp.float32),
    )(edge_attr, We, be.reshape(L, 1, H))


def _outffn_body(h_ref, m0_ref, m1_ref, wo_ref, bo_ref, g1_ref, b1_ref,
                 w1_ref, bf1_ref, w2_ref, bf2_ref, g2_ref, b2_ref, o_ref):
    m0 = m0_ref[...]
    m1 = m1_ref[...]
    msg = jnp.concatenate([m0[0], m1[0], m0[1], m1[1]], axis=-1)
    h1 = _ln(h_ref[...] + _dot(msg, wo_ref[...]) + bo_ref[...],
             g1_ref[...], b1_ref[...])
    ffn = _dot(jax.nn.gelu(_dot(h1, w1_ref[...]) + bf1_ref[...]),
               w2_ref[...]) + bf2_ref[...]
    o_ref[...] = _ln(h1 + ffn, g2_ref[...], b2_ref[...])


def _outffn(h, msg0, msg1, Wo, bo, g1, b1, W1, bf1, W2, bf2, g2, b2):
    r1 = lambda a, n: a.reshape(1, n)
    return pl.pallas_call(
        _outffn_body,
        grid=(NB,),
        in_specs=[
            pl.BlockSpec((BR, D), lambda i: (i, 0)),
            pl.BlockSpec((NC, BR, DP), lambda i: (0, i, 0)),
            pl.BlockSpec((NC, BR, DP), lambda i: (0, i, 0)),
            pl.BlockSpec((D, D), lambda i: (0, 0)),
            pl.BlockSpec((1, D), lambda i: (0, 0)),
            pl.BlockSpec((1, D), lambda i: (0, 0)),
            pl.BlockSpec((1, D), lambda i: (0, 0)),
            pl.BlockSpec((D, DFF), lambda i: (0, 0)),
            pl.BlockSpec((1, DFF), lambda i: (0, 0)),
            pl.BlockSpec((DFF, D), lambda i: (0, 0)),
            pl.BlockSpec((1, D), lambda i: (0, 0)),
            pl.BlockSpec((1, D), lambda i: (0, 0)),
            pl.BlockSpec((1, D), lambda i: (0, 0)),
        ],
        out_specs=pl.BlockSpec((BR, D), lambda i: (i, 0)),
        out_shape=jax.ShapeDtypeStruct((N, D), jnp.float32),
    )(h, msg0.reshape(NC, N, DP), msg1.reshape(NC, N, DP), Wo, r1(bo, D),
      r1(g1, D), r1(b1, D), W1, r1(bf1, DFF), W2, r1(bf2, D), r1(g2, D),
      r1(b2, D))


def _pool_body(h_ref, batch_ref, wh_ref, bh_ref, wm_ref, bm_ref,
               pred_ref, mean_ref, sums, cnts):
    i = pl.program_id(0)

    @pl.when(i == 0)
    def _():
        sums[...] = jnp.zeros_like(sums)
        cnts[...] = jnp.zeros_like(cnts)

    b = batch_ref[0, 0]
    mask = (b[:, None] == lax.broadcasted_iota(jnp.int32, (BR, NG), 1)
            ).astype(jnp.float32)
    sums[...] += lax.dot_general(mask, h_ref[...], (((0,), (0,)), ((), ())),
                                 preferred_element_type=jnp.float32)
    cnts[...] += lax.dot_general(mask, jnp.ones((BR, D), jnp.float32),
                                 (((0,), (0,)), ((), ())),
                                 preferred_element_type=jnp.float32)

    @pl.when(i == NB - 1)
    def _():
        x_mean = sums[...] / jnp.maximum(cnts[...], 1.0)
        hh = jax.nn.gelu(_dot(x_mean, wh_ref[...]) + bh_ref[...])
        pred_ref[...] = _dot(hh, wm_ref[...]) + bm_ref[...]
        mean_ref[...] = x_mean


def _pool_head(h, batch, Wh1, bh1, Wml, bml):
    return pl.pallas_call(
        _pool_body,
        grid=(NB,),
        in_specs=[
            pl.BlockSpec((BR, D), lambda i: (i, 0)),
            pl.BlockSpec((1, 1, BR), lambda i: (i, 0, 0)),
            pl.BlockSpec((D, D // 2), lambda i: (0, 0)),
            pl.BlockSpec((1, D // 2), lambda i: (0, 0)),
            pl.BlockSpec((D // 2, 2), lambda i: (0, 0)),
            pl.BlockSpec((1, 2), lambda i: (0, 0)),
        ],
        out_specs=[
            pl.BlockSpec((NG, 2), lambda i: (0, 0)),
            pl.BlockSpec((NG, D), lambda i: (0, 0)),
        ],
        out_shape=[
            jax.ShapeDtypeStruct((NG, 2), jnp.float32),
            jax.ShapeDtypeStruct((NG, D), jnp.float32),
        ],
        scratch_shapes=[
            pltpu.VMEM((NG, D), jnp.float32),
            pltpu.VMEM((NG, D), jnp.float32),
        ],
    )(h, batch.reshape(NB, 1, BR), Wh1, bh1.reshape(1, D // 2),
      Wml, bml.reshape(1, 2))


def _sc_mesh():
    return plsc.VectorSubcoreMesh(core_axis_name="c", subcore_axis_name="s",
                                  num_cores=NC, num_subcores=NS)


def _iota16():
    return lax.iota(jnp.int32, 16)


def _full16(x):
    return jnp.zeros((16,), jnp.int32) + x


def _edge_chunk_compute(qd, kvs, ebb, exv, c, p):
    iota = _iota16()

    def _group(g, _):
        rows = g * 16 + iota

        def _qk(j, acc):
            a00, a01, a10, a11 = acc
            c00 = (iota + j) & (DH - 1)
            c01 = (iota + (DH // 2 + j)) & (DH - 1)
            c10 = DH + c00
            c11 = DH + c01
            a00 = a00 + (plsc.load_gather(qd, [rows, c00])
                         * plsc.load_gather(kvs, [rows, c00]))
            a01 = a01 + (plsc.load_gather(qd, [rows, c01])
                         * plsc.load_gather(kvs, [rows, c01]))
            a10 = a10 + (plsc.load_gather(qd, [rows, c10])
                         * plsc.load_gather(kvs, [rows, c10]))
            a11 = a11 + (plsc.load_gather(qd, [rows, c11])
                         * plsc.load_gather(kvs, [rows, c11]))
            return a00, a01, a10, a11

        zf = jnp.zeros((16,), jnp.float32)
        a00, a01, a10, a11 = plsc.parallel_loop(
            0, DH // 2, unroll=8, carry=(zf, zf, zf, zf))(_qk)
        ebbase = c * (NP * HP) + p * HP
        eb0 = plsc.load_gather(ebb, [rows, _full16(ebbase)])
        eb1 = plsc.load_gather(ebb, [rows, _full16(ebbase + 1)])
        ex0 = jnp.exp((a00 + a01) * INV_SQRT_DH + eb0)
        ex1 = jnp.exp((a10 + a11) * INV_SQRT_DH + eb1)
        plsc.store_scatter(exv, [rows, _full16(DP)], ex0)
        plsc.store_scatter(exv, [rows, _full16(DP + 1)], ex1)

        def _ev(j):
            c0 = (iota + j) & (DH - 1)
            c1 = DH + c0
            v0 = plsc.load_gather(kvs, [rows, _full16(DP) + c0])
            v1 = plsc.load_gather(kvs, [rows, _full16(DP) + c1])
            plsc.store_scatter(exv, [rows, c0], v0 * ex0)
            plsc.store_scatter(exv, [rows, c1], v1 * ex1)

        plsc.parallel_loop(0, DH, unroll=8)(_ev)
        return 0

    lax.fori_loop(0, CB // 16, _group, 0)


def _edge_kernel(p, q_hbm, kv_hbm, src_hbm, dst_hbm, eb_hbm, out_hbm,
                 msgacc,
                 srcA, dstrA, dstaA, qdA, kvsA, ebA, exvA,
                 srcB, dstrB, dstaB, qdB, kvsB, ebB, exvB,
                 srcC, dstrC, dstaC, qdC, kvsC, ebC, exvC,
                 zbuf, macc, obuf, macc2, obuf2,
                 gA, gB, gC, iA, iB, iC, scA, scB, scC):
    c = lax.axis_index("c")
    s = lax.axis_index("s")
    qoff = (c * NP + p) * N
    iota = _iota16()
    zf = jnp.zeros((16,), jnp.float32)

    def _zrow(i, _):
        for j in range(ACC_W // 16):
            plsc.store_scatter(zbuf, [_full16(i), _full16(j * 16) + iota], zf)
        return 0
    lax.fori_loop(0, 16, _zrow, 0)

    def _zacc(t, _):
        pltpu.async_copy(zbuf, msgacc.at[pl.ds(s * 624 + t * 16, 16)], gA)
        return 0
    lax.fori_loop(0, 39, _zacc, 0)

    @pl.when(s == NS - 1)
    def _():
        pltpu.async_copy(zbuf, msgacc.at[pl.ds(9984, 16)], gA)

    def _zacc_wait(t, _):
        pltpu.make_async_copy(zbuf, msgacc.at[pl.ds(s * 624, 16)], gA).wait()
        return 0
    lax.fori_loop(0, 39, _zacc_wait, 0)

    @pl.when(s == NS - 1)
    def _():
        pltpu.make_async_copy(zbuf, msgacc.at[pl.ds(9984, 16)], gA).wait()

    def _zpad(exv):
        def body(r, _):
            plsc.store_scatter(exv, [_full16(r), _full16(DP) + iota], zf,
                               mask=iota >= HP)
            return 0
        lax.fori_loop(0, CB, body, 0)
    _zpad(exvA)
    _zpad(exvB)
    _zpad(exvC)

    plsc.subcore_barrier()

    e0 = s * EPT
    LAST = NCHUNK - 1
    sets = (
        (srcA, dstrA, dstaA, qdA, kvsA, ebA, exvA, gA, iA, scA),
        (srcB, dstrB, dstaB, qdB, kvsB, ebB, exvB, gB, iB, scB),
        (srcC, dstrC, dstaC, qdC, kvsC, ebC, exvC, gC, iC, scC),
    )

    def _issue_idx(t, st):
        (srcv, dstr, dsta, qd, kvs, eb, exv, g, si, sc) = st
        base = e0 + t * CB
        pltpu.async_copy(src_hbm.at[pl.ds(base, CB)], srcv, si)
        pltpu.async_copy(dst_hbm.at[pl.ds(base, CB)], dstr, si)
        pltpu.async_copy(eb_hbm.at[pl.ds(base, CB)], eb, si)

    def _wait_idx(t, st):
        (srcv, dstr, dsta, qd, kvs, eb, exv, g, si, sc) = st
        base = e0 + t * CB
        pltpu.make_async_copy(src_hbm.at[pl.ds(base, CB)], srcv, si).wait()
        pltpu.make_async_copy(dst_hbm.at[pl.ds(base, CB)], dstr, si).wait()
        pltpu.make_async_copy(eb_hbm.at[pl.ds(base, CB)], eb, si).wait()

    def _issue_gathers(st):
        (srcv, dstr, dsta, qd, kvs, eb, exv, g, si, sc) = st
        for gi in range(CB // 16):
            sl = pl.ds(gi * 16, 16)
            dsta[sl] = dstr[sl] + qoff
            srcv[sl] = srcv[sl] + qoff
        pltpu.async_copy(q_hbm.at[dsta], qd, g)
        pltpu.async_copy(kv_hbm.at[srcv], kvs, g)

    def _wait_gathers(st):
        (srcv, dstr, dsta, qd, kvs, eb, exv, g, si, sc) = st
        pltpu.make_async_copy(q_hbm.at[dsta], qd, g).wait()
        pltpu.make_async_copy(kv_hbm.at[srcv], kvs, g).wait()

    def _wait_scatter(st):
        (srcv, dstr, dsta, qd, kvs, eb, exv, g, si, sc) = st
        pltpu.make_async_copy(exv, msgacc.at[dstr], sc).wait()

    def _step(t, sc_cur, sn1, sn2):
        @pl.when(t + 1 <= LAST)
        def _():
            _wait_idx(t + 1, sn1)
            _issue_gathers(sn1)

        @pl.when(jnp.logical_and(t >= 1, t + 2 <= LAST))
        def _():
            _wait_scatter(sn2)

        @pl.when(t + 2 <= LAST)
        def _():
            _issue_idx(t + 2, sn2)

        st = sc_cur
        _wait_gathers(st)
        (srcv, dstr, dsta, qd, kvs, eb, exv, g, si, sc) = st
        _edge_chunk_compute(qd, kvs, eb, exv, c, p)
        pltpu.async_copy(exv, msgacc.at[dstr], sc, add=True)

    _issue_idx(0, sets[0])
    _wait_idx(0, sets[0])
    _issue_gathers(sets[0])
    _issue_idx(1, sets[1])

    def _pipe(i, _):
        t0 = 3 * i
        _step(t0, sets[0], sets[1], sets[2])
        _step(t0 + 1, sets[1], sets[2], sets[0])
        _step(t0 + 2, sets[2], sets[0], sets[1])
        return 0

    lax.fori_loop(0, NCHUNK // 3, _pipe, 0)
    for r in range(NCHUNK - (NCHUNK // 3) * 3):
        t = (NCHUNK // 3) * 3 + r
        _step(t, sets[t % 3], sets[(t + 1) % 3], sets[(t + 2) % 3])

    for st in sets:
        _wait_scatter(st)

    plsc.subcore_barrier()

    nblk = 39 + jnp.where(s == NS - 1, 1, 0)

    def _r0(b):
        return jnp.where(b < 39, s * 624 + b * 16, 9984)

    pars = ((macc, obuf, iA, scA), (macc2, obuf2, iB, scB))

    def _nstep(b, cur, nxt):
        (macc_c, obuf_c, si_c, so_c) = cur
        (macc_n, obuf_n, si_n, so_n) = nxt

        @pl.when(b + 1 < nblk)
        def _():
            pltpu.async_copy(msgacc.at[pl.ds(_r0(b + 1), 16)], macc_n, si_n)

        pltpu.make_async_copy(msgacc.at[pl.ds(0, 16)], macc_c, si_c).wait()

        @pl.when(b >= 2)
        def _():
            pltpu.make_async_copy(obuf_c, out_hbm.at[pl.ds(0, 16)],
                                  so_c).wait()

        def _row(r, _):
            rr = _full16(r)
            for j in range(DP // 16):
                den = plsc.load_gather(macc_c, [rr, _full16(DP + j // 2)])
                val = plsc.load_gather(macc_c, [rr, _full16(j * 16) + iota])
                plsc.store_scatter(obuf_c, [rr, _full16(j * 16) + iota],
                                   val / (den + 1e-9))
            return 0
        lax.fori_loop(0, 16, _row, 0)
        pltpu.async_copy(obuf_c, out_hbm.at[pl.ds(c * N + _r0(b), 16)], so_c)

    pltpu.async_copy(msgacc.at[pl.ds(_r0(0), 16)], macc, iA)

    def _npair(k, _):
        b0 = 2 * k

        @pl.when(b0 < nblk)
        def _():
            _nstep(b0, pars[0], pars[1])

        @pl.when(b0 + 1 < nblk)
        def _():
            _nstep(b0 + 1, pars[1], pars[0])
        return 0
    lax.fori_loop(0, 20, _npair, 0)

    for (_m, obuf_c, _si, so_c) in pars:
        pltpu.make_async_copy(obuf_c, out_hbm.at[pl.ds(0, 16)], so_c).wait()


def _edge_attention(p, q4, kv4, src, dst, eb_l):
    f32 = jnp.float32
    scratch = [pltpu.VMEM_SHARED((N, ACC_W), f32)]
    for _ in range(3):
        scratch += [
            pltpu.VMEM((CB,), jnp.int32),
            pltpu.VMEM((CB,), jnp.int32),
            pltpu.VMEM((CB,), jnp.int32),
            pltpu.VMEM((CB, DP), f32),
            pltpu.VMEM((CB, 2 * DP), f32),
            pltpu.VMEM((CB, H), f32),
            pltpu.VMEM((CB, ACC_W), f32),
        ]
    scratch += [
        pltpu.VMEM((16, ACC_W), f32),
        pltpu.VMEM((16, ACC_W), f32),
        pltpu.VMEM((16, DP), f32),
        pltpu.VMEM((16, ACC_W), f32),
        pltpu.VMEM((16, DP), f32),
    ] + [pltpu.SemaphoreType.DMA] * 9

    run = pl.kernel(
        functools.partial(_edge_kernel, p),
        out_type=jax.ShapeDtypeStruct((NC * N, DP), f32),
        mesh=_sc_mesh(),
        scratch_types=scratch,
        compiler_params=pltpu.CompilerParams(use_tc_tiling_on_sc=False,
                                             needs_layout_passes=False),
    )
    return run(q4, kv4, src, dst, eb_l)


def _quarters(W):
    return W.reshape(D, NC * NP, DP).transpose(1, 0, 2)


def kernel(x, edge_index, batch, edge_attr, Win, b_in, ln0_g, ln0_b,
           Wq, Wk, Wv, Wo, bo, We, be, ln1_g, ln1_b,
           W1, b1, W2, b2, ln2_g, ln2_b,
           Wh1, bh1, Wm, bm, Wlv, blv):
    src = edge_index[0]
    dst = edge_index[1]

    h = _inproj(x, Win, b_in, ln0_g, ln0_b)
    eb_all = _edge_bias(edge_attr, We, be)

    for l in range(L):
        q4, kv4 = _qkv(h, _quarters(Wq[l]), _quarters(Wk[l]),
                       _quarters(Wv[l]))
        msg0 = _edge_attention(0, q4, kv4, src, dst, eb_all[l])
        msg1 = _edge_attention(1, q4, kv4, src, dst, eb_all[l])
        h = _outffn(h, msg0, msg1, Wo[l], bo[l],
                    ln1_g[l], ln1_b[l], W1[l], b1[l], W2[l], b2[l],
                    ln2_g[l], ln2_b[l])

    Wml = jnp.concatenate([Wm, Wlv], axis=1)
    bml = jnp.concatenate([bm, blv], axis=0)
    pred, x_mean = _pool_head(h, batch, Wh1, bh1, Wml, bml)
    return (pred, x_mean)
